# Initial kernel scaffold; baseline (speedup 1.0000x reference)
import jax
import jax.numpy as jnp
from jax.experimental import pallas as pl


def kernel(x, edge_index, W1, b1, W2, b2):
    raise NotImplementedError("write your pallas kernel here")


# speedup vs baseline: 10.5422x; 10.5422x over previous
import functools

import jax
import jax.numpy as jnp
from jax import lax
from jax.experimental import pallas as pl
from jax.experimental.pallas import tpu as pltpu
from jax.experimental.pallas import tpu_sc as plsc

N = 10000
F = 128
NC, NS = 2, 16
NW = NC * NS
CHUNK = 128
RPT = 640
N_PAD = NS * RPT
DW = 128

_MESH = plsc.VectorSubcoreMesh(core_axis_name="c", subcore_axis_name="s")


def _sc_degree(edges, zeros_col, ones_col, ept):
    nchunk = ept // CHUNK

    @functools.partial(
        pl.kernel,
        out_type=jax.ShapeDtypeStruct((NC, N_PAD, DW), jnp.float32),
        mesh=_MESH,
        scratch_types=[
            pltpu.VMEM((CHUNK,), jnp.int32),
            pltpu.VMEM((CHUNK, DW), jnp.float32),
            pltpu.VMEM_SHARED((N_PAD, DW), jnp.float32),
        ],
    )
    def k(e_hbm, z_hbm, o_hbm, out_hbm, dst_idx, ones_v, acc):
        c = lax.axis_index("c")
        s = lax.axis_index("s")
        pltpu.sync_copy(z_hbm, acc.at[pl.ds(s * RPT, RPT)])
        pltpu.sync_copy(o_hbm, ones_v)
        plsc.subcore_barrier()
        e0 = (c * NS + s) * ept

        def step(j, carry):
            base = e0 + j * CHUNK
            pltpu.sync_copy(e_hbm.at[1, pl.ds(base, CHUNK)], dst_idx)
            pltpu.sync_copy(ones_v, acc.at[dst_idx], add=True)
            return carry

        lax.fori_loop(0, nchunk, step, 0)
        plsc.subcore_barrier()
        pltpu.sync_copy(acc.at[pl.ds(s * RPT, RPT)],
                        out_hbm.at[c, pl.ds(s * RPT, RPT)])

    return k(edges, zeros_col, ones_col)


def _sc_aggregate(y, edges, zeros_blk, ept):
    nchunk = ept // CHUNK

    @functools.partial(
        pl.kernel,
        out_type=jax.ShapeDtypeStruct((NC, N_PAD, F), jnp.float32),
        mesh=_MESH,
        scratch_types=[
            pltpu.VMEM((CHUNK,), jnp.int32),
            pltpu.VMEM((CHUNK,), jnp.int32),
            pltpu.VMEM((CHUNK, F), jnp.float32),
            pltpu.VMEM_SHARED((N_PAD, F), jnp.float32),
            pltpu.SemaphoreType.DMA,
        ],
    )
    def k(y_hbm, e_hbm, z_hbm, out_hbm, src_idx, dst_idx, rows, acc, sem):
        c = lax.axis_index("c")
        s = lax.axis_index("s")
        pltpu.sync_copy(z_hbm, acc.at[pl.ds(s * RPT, RPT)])
        plsc.subcore_barrier()
        e0 = (c * NS + s) * ept

        def step(j, carry):
            base = e0 + j * CHUNK
            pltpu.sync_copy(e_hbm.at[0, pl.ds(base, CHUNK)], src_idx)
            pltpu.sync_copy(e_hbm.at[1, pl.ds(base, CHUNK)], dst_idx)
            pltpu.async_copy(y_hbm.at[src_idx], rows, sem).wait()
            pltpu.sync_copy(rows, acc.at[dst_idx], add=True)
            return carry

        lax.fori_loop(0, nchunk, step, 0)
        plsc.subcore_barrier()
        pltpu.sync_copy(acc.at[pl.ds(s * RPT, RPT)],
                        out_hbm.at[c, pl.ds(s * RPT, RPT)])

    return k(y, edges, zeros_blk)


_BR = 2000
_GRID = (N + _BR - 1) // _BR


def _tc_first(x, W, degp):
    def body(x_ref, w_ref, d0_ref, d1_ref, y_ref, dis_ref):
        deg = d0_ref[0][:, 0:1] + d1_ref[0][:, 0:1] + 1.0
        dis = lax.rsqrt(deg)
        xw = lax.dot_general(x_ref[...], w_ref[...],
                             (((1,), (1,)), ((), ())),
                             preferred_element_type=jnp.float32)
        y_ref[...] = xw * dis
        dis_ref[...] = dis

    return pl.pallas_call(
        body,
        grid=(_GRID,),
        in_specs=[
            pl.BlockSpec((_BR, F), lambda i: (i, 0)),
            pl.BlockSpec((F, F), lambda i: (0, 0)),
            pl.BlockSpec((1, _BR, DW), lambda i: (0, i, 0)),
            pl.BlockSpec((1, _BR, DW), lambda i: (1, i, 0)),
        ],
        out_specs=[
            pl.BlockSpec((_BR, F), lambda i: (i, 0)),
            pl.BlockSpec((_BR, 1), lambda i: (i, 0)),
        ],
        out_shape=[
            jax.ShapeDtypeStruct((N, F), jnp.float32),
            jax.ShapeDtypeStruct((N, 1), jnp.float32),
        ],
    )(x, W, degp, degp)


def _tc_mid(parts, y1, dis, b1, W2):
    def body(p0_ref, p1_ref, y1_ref, dis_ref, b_ref, w_ref, y2_ref):
        dis = dis_ref[...]
        h = (p0_ref[0] + p1_ref[0] + y1_ref[...]) * dis + b_ref[...]
        hw = lax.dot_general(h, w_ref[...], (((1,), (1,)), ((), ())),
                             preferred_element_type=jnp.float32)
        y2_ref[...] = hw * dis

    return pl.pallas_call(
        body,
        grid=(_GRID,),
        in_specs=[
            pl.BlockSpec((1, _BR, F), lambda i: (0, i, 0)),
            pl.BlockSpec((1, _BR, F), lambda i: (1, i, 0)),
            pl.BlockSpec((_BR, F), lambda i: (i, 0)),
            pl.BlockSpec((_BR, 1), lambda i: (i, 0)),
            pl.BlockSpec((1, F), lambda i: (0, 0)),
            pl.BlockSpec((F, F), lambda i: (0, 0)),
        ],
        out_specs=pl.BlockSpec((_BR, F), lambda i: (i, 0)),
        out_shape=jax.ShapeDtypeStruct((N, F), jnp.float32),
    )(parts, parts, y1, dis, b1, W2)


def _tc_last(parts, y2, dis, b2):
    def body(p0_ref, p1_ref, y2_ref, dis_ref, b_ref, out_ref):
        out_ref[...] = ((p0_ref[0] + p1_ref[0] + y2_ref[...])
                        * dis_ref[...] + b_ref[...])

    return pl.pallas_call(
        body,
        grid=(_GRID,),
        in_specs=[
            pl.BlockSpec((1, _BR, F), lambda i: (0, i, 0)),
            pl.BlockSpec((1, _BR, F), lambda i: (1, i, 0)),
            pl.BlockSpec((_BR, F), lambda i: (i, 0)),
            pl.BlockSpec((_BR, 1), lambda i: (i, 0)),
            pl.BlockSpec((1, F), lambda i: (0, 0)),
        ],
        out_specs=pl.BlockSpec((_BR, F), lambda i: (i, 0)),
        out_shape=jax.ShapeDtypeStruct((N, F), jnp.float32),
    )(parts, parts, y2, dis, b2)


def kernel(x, edge_index, W1, b1, W2, b2):
    E = edge_index.shape[1]
    ept = ((E + NW - 1) // NW + CHUNK - 1) // CHUNK * CHUNK
    e_pad = ept * NW
    e = edge_index.astype(jnp.int32)
    pad = e_pad - E
    src = jnp.concatenate([e[0], jnp.zeros((pad,), jnp.int32)])
    dst = jnp.concatenate([e[1], jnp.full((pad,), N, jnp.int32)])
    edges = jnp.stack([src, dst])

    zeros_blk = jnp.zeros((RPT, F), jnp.float32)
    zeros_col = jnp.zeros((RPT, DW), jnp.float32)
    ones_col = jnp.ones((CHUNK, DW), jnp.float32)
    b1r = b1.reshape(1, F)
    b2r = b2.reshape(1, F)

    degp = _sc_degree(edges, zeros_col, ones_col, ept)
    y1, dis = _tc_first(x, W1, degp)
    s1 = _sc_aggregate(y1, edges, zeros_blk, ept)
    y2 = _tc_mid(s1, y1, dis, b1r, W2)
    s2 = _sc_aggregate(y2, edges, zeros_blk, ept)
    return _tc_last(s2, y2, dis, b2r)
